# Initial kernel scaffold; baseline (speedup 1.0000x reference)
import jax
import jax.numpy as jnp
from jax.experimental import pallas as pl


def kernel(emb, weights, W, b, indexes, h_offset, w_offset, img_h, img_w):
    raise NotImplementedError("write your pallas kernel here")


# speedup vs baseline: 1.7528x; 1.7528x over previous
import functools

import jax
import jax.numpy as jnp
from jax import lax
from jax.experimental import pallas as pl
from jax.experimental.pallas import tpu as pltpu
from jax.experimental.pallas import tpu_sc as plsc

_GRID_W = 128
_NB = 8
_A = 4096
_CE = 256
_PD = 48
_PROWS = _A + 1
_TRASH = _A


def _tc_body(emb_ref, wvec_ref, w_ref, b_ref, out_ref):
    x = emb_ref[0]
    res = jnp.dot(x, w_ref[...], preferred_element_type=jnp.float32)
    res = (res + b_ref[...]) * wvec_ref[0]
    z4 = jnp.zeros((_A, 4), jnp.float32)
    res64 = jnp.concatenate(
        [res[:, 0:12], z4, res[:, 12:24], z4,
         res[:, 24:36], z4, res[:, 36:48], z4], axis=1)
    out_ref[0, :_A, :] = res64
    out_ref[0, _A:_PROWS, :] = jnp.zeros((1, 64), jnp.float32)


def _tc_compute(emb, wvec, w, bvec):
    return pl.pallas_call(
        _tc_body,
        grid=(_NB,),
        in_specs=[
            pl.BlockSpec((1, _A, _CE), lambda i: (i, 0, 0)),
            pl.BlockSpec((1, _A, 1), lambda i: (i, 0, 0)),
            pl.BlockSpec((_CE, _PD), lambda i: (0, 0)),
            pl.BlockSpec((1, _PD), lambda i: (0, 0)),
        ],
        out_specs=pl.BlockSpec((1, _PROWS, 64), lambda i: (i, 0, 0)),
        out_shape=jax.ShapeDtypeStruct((_NB, _PROWS, 64), jnp.float32),
    )(emb, wvec, w, bvec)


def _sc_body(p_hbm, idx_hbm, wts_hbm, offs_hbm, img_out, wimg_out,
             idx_v, wts_v, offs_v, tbl_v, wvals_v, avals_v, idxg_v,
             gbuf_v, obuf_v, wimg_v, sem):
    c = lax.axis_index("c")
    s = lax.axis_index("s")
    wid = s * 2 + c
    b = wid // 4
    q = wid % 4

    pltpu.sync_copy(idx_hbm.at[b], idx_v)
    pltpu.sync_copy(wts_hbm.at[b], wts_v)
    pltpu.sync_copy(offs_hbm, offs_v)

    lanes = lax.iota(jnp.int32, 16)
    bb = jnp.full((16,), b, jnp.int32)
    hoff = plsc.load_gather(offs_v, [bb])
    woff = plsc.load_gather(offs_v, [bb + 8])
    row0 = hoff + q * 16

    def _init(i, carry):
        tbl_v[pl.ds(i * 16, 16)] = jnp.full((16,), _TRASH, jnp.int32)
        return carry
    lax.fori_loop(0, 128, _init, 0)

    def _scat(i, carry):
        iv = idx_v[pl.ds(i * 16, 16)]
        gh = iv >> 7
        gw = iv & (_GRID_W - 1)
        lr = gh - row0
        valid = (lr >= 0) & (lr < 16)
        tix = jnp.where(valid, lr * _GRID_W + gw, 0)
        plsc.store_scatter(tbl_v, [tix], lanes + i * 16, mask=valid)
        return carry
    lax.fori_loop(0, 256, _scat, 0)

    def _patches(i, carry):
        rl = i // 4
        ch = i % 4
        tix = rl * _GRID_W + woff + ch * 16 + lanes
        a = plsc.load_gather(tbl_v, [tix])
        wv = plsc.load_gather(wts_v, [jnp.minimum(a, _A - 1)])
        wv = jnp.where(a < _A, wv, 0.0)
        base = rl * 64 + ch * 16
        wvals_v[pl.ds(base, 16)] = wv
        avals_v[pl.ds(base, 16)] = a
        return carry
    lax.fori_loop(0, 64, _patches, 0)

    pbase = b * (_PROWS * 4)
    def _mkidx(j, carry):
        for hc in range(8):
            p0 = j * 128 + hc * 16
            line = p0 // 64
            rl = line // 4
            r = line % 4
            a = avals_v[pl.ds((rl * 64) + (p0 % 64), 16)]
            idxg_v[j, pl.ds(hc * 16, 16)] = pbase + a * 4 + r
        return carry
    lax.fori_loop(0, 32, _mkidx, 0)

    mask12 = lax.iota(jnp.int32, 16) < 12

    def _quarter(q4, carry):
        def _fire(j, carry2):
            pltpu.async_copy(p_hbm.at[idxg_v.at[q4 * 8 + j]],
                             gbuf_v.at[j], sem)
            return carry2
        lax.fori_loop(0, 8, _fire, 0)

        def _drain(j, carry2):
            pltpu.make_async_copy(p_hbm.at[idxg_v.at[q4 * 8 + j]],
                                  gbuf_v.at[j], sem).wait()
            return carry2
        lax.fori_loop(0, 8, _drain, 0)

        def _compact(t, carry2):
            x = gbuf_v[t // 128, t % 128]
            plsc.store_compressed(obuf_v.at[pl.ds(t * 12, 16)], x,
                                  mask=mask12)
            return carry2
        lax.fori_loop(0, 1024, _compact, 0)
        pltpu.sync_copy(obuf_v.at[pl.ds(0, 12288)], img_out.at[wid, q4])
        return carry
    lax.fori_loop(0, 4, _quarter, 0)

    def _wimg(i, carry):
        rl = i // 16
        ci = i % 16
        widx = rl * 64 + ((ci * 16 + lanes) >> 2)
        wvv = plsc.load_gather(wvals_v, [widx])
        for r in range(4):
            wimg_v[pl.ds((rl * 4 + r) * 256 + ci * 16, 16)] = wvv
        return carry
    lax.fori_loop(0, 256, _wimg, 0)
    pltpu.sync_copy(wimg_v, wimg_out.at[wid])


@functools.partial(
    pl.kernel,
    out_type=(
        jax.ShapeDtypeStruct((32, 4, 12288), jnp.float32),
        jax.ShapeDtypeStruct((32, 16384), jnp.float32),
    ),
    mesh=plsc.VectorSubcoreMesh(core_axis_name="c", subcore_axis_name="s"),
    compiler_params=pltpu.CompilerParams(needs_layout_passes=False,
                                         use_tc_tiling_on_sc=False),
    scratch_types=(
        pltpu.VMEM((_A,), jnp.int32),
        pltpu.VMEM((_A,), jnp.float32),
        pltpu.VMEM((128,), jnp.int32),
        pltpu.VMEM((16 * _GRID_W,), jnp.int32),
        pltpu.VMEM((1024,), jnp.float32),
        pltpu.VMEM((1024,), jnp.int32),
        pltpu.VMEM((32, 128), jnp.int32),
        pltpu.VMEM((8, 128, 16), jnp.float32),
        pltpu.VMEM((12288 + 16,), jnp.float32),
        pltpu.VMEM((16384,), jnp.float32),
        pltpu.SemaphoreType.DMA,
    ),
)
def _sc_kernel(p_hbm, idx_hbm, wts_hbm, offs_hbm, img_out, wimg_out,
               idx_v, wts_v, offs_v, tbl_v, wvals_v, avals_v, idxg_v,
               gbuf_v, obuf_v, wimg_v, sem):
    _sc_body(p_hbm, idx_hbm, wts_hbm, offs_hbm, img_out, wimg_out,
             idx_v, wts_v, offs_v, tbl_v, wvals_v, avals_v, idxg_v,
             gbuf_v, obuf_v, wimg_v, sem)


def kernel(emb, weights, W, b, indexes, h_offset, w_offset, img_h, img_w):
    del img_h, img_w
    p = _tc_compute(emb, weights.reshape(_NB, _A, 1), W, b.reshape(1, _PD))
    p_rows = p.reshape(_NB * _PROWS * 4, 16)
    offs = jnp.concatenate([h_offset.astype(jnp.int32),
                            w_offset.astype(jnp.int32),
                            jnp.zeros((112,), jnp.int32)])
    img4, wimg4 = _sc_kernel(p_rows, indexes, weights, offs)
    img = img4.reshape(_NB, 256, 256, 3)
    wimg = wimg4.reshape(_NB, 256, 256, 1)
    return img, wimg
